# Initial kernel scaffold; baseline (speedup 1.0000x reference)
import jax
import jax.numpy as jnp
from jax.experimental import pallas as pl


def kernel(input, word_table):
    raise NotImplementedError("write your pallas kernel here")


# speedup vs baseline: 1.5814x; 1.5814x over previous
import functools

import jax
import jax.numpy as jnp
import numpy as np
from jax import lax
from jax.experimental import pallas as pl
from jax.experimental.pallas import tpu as pltpu
from jax.experimental.pallas import tpu_sc as plsc

D_MODEL = 768
MAX_LEN = 8192
LANES = 16
NC, NS = 2, 16
NW = NC * NS


def _pe_table(max_len, d_model):
    pos = np.arange(max_len, dtype=np.float64)[:, None]
    i = np.arange(0, d_model, 2, dtype=np.float64)[None, :]
    angle = pos / np.power(10000.0, i / d_model)
    pe = np.zeros((max_len, d_model), dtype=np.float32)
    pe[:, 0::2] = np.sin(angle).astype(np.float32)
    pe[:, 1::2] = np.cos(angle).astype(np.float32)
    return pe


_PE = _pe_table(MAX_LEN, D_MODEL)


@functools.partial(jax.jit, static_argnames=("batch", "seq"))
def _sc_embed(idx_flat, word_table, pe, *, batch, seq):
    p_per_w = seq // NW
    chunk = 32
    n_ch = p_per_w // chunk
    kv = D_MODEL // LANES

    mesh = plsc.VectorSubcoreMesh(core_axis_name="c", subcore_axis_name="s")

    @functools.partial(
        pl.kernel,
        mesh=mesh,
        out_type=jax.ShapeDtypeStruct((batch * seq, D_MODEL), jnp.float32),
        scratch_types=[
            pltpu.VMEM((p_per_w, D_MODEL), jnp.float32),
            pltpu.VMEM((chunk,), jnp.int32),
            pltpu.VMEM((chunk, D_MODEL), jnp.float32),
            pltpu.SemaphoreType.DMA,
        ],
    )
    def k(idx_hbm, table_hbm, pe_hbm, out_hbm, pe_v, idx_v, rows_v, sem):
        wid = lax.axis_index("s") * NC + lax.axis_index("c")
        pbase = wid * p_per_w
        pltpu.sync_copy(pe_hbm.at[pl.ds(pbase, p_per_w)], pe_v)
        for b in range(batch):
            for ci in range(n_ch):
                start = b * seq + pbase + ci * chunk
                pltpu.sync_copy(idx_hbm.at[pl.ds(start, chunk)], idx_v)
                pltpu.async_copy(table_hbm.at[idx_v], rows_v, sem).wait()

                def body(r, carry, ci=ci):
                    for kk in range(kv):
                        v = pe_v[ci * chunk + r, pl.ds(kk * LANES, LANES)]
                        plsc.addupdate(rows_v.at[r, pl.ds(kk * LANES, LANES)], v)
                    return carry

                lax.fori_loop(0, chunk, body, 0)
                pltpu.sync_copy(rows_v, out_hbm.at[pl.ds(start, chunk)])

    return k(idx_flat, word_table, pe)


def kernel(input, word_table):
    batch, seq = input.shape
    idx = input.reshape(-1).astype(jnp.int32)
    pe = jnp.asarray(_PE[:seq])
    out = _sc_embed(idx, word_table, pe, batch=batch, seq=seq)
    return out.reshape(batch, seq, D_MODEL)
